# Initial kernel scaffold; baseline (speedup 1.0000x reference)
import jax
import jax.numpy as jnp
from jax.experimental import pallas as pl


def kernel(hidden_states, position_ids, Wq, Wk, Wv, Wo):
    raise NotImplementedError("write your pallas kernel here")


# speedup vs baseline: 1.4276x; 1.4276x over previous
import jax
import jax.numpy as jnp
from jax.experimental import pallas as pl

_B, _S, _HID, _NH = 1, 2048, 768, 12
_HD = _HID // _NH
_THETA = 10000.0
_SCALE = 1.0 / (_HD ** 0.5)
_NEG = float(jnp.finfo(jnp.float32).min)


def _attn_head_kernel(x_ref, cos_ref, sin_ref, wq_ref, wk_ref, wv_ref, wo_ref,
                      out_ref):
    h = pl.program_id(0)
    x = x_ref[...]
    cos = cos_ref[...]
    sin = sin_ref[...]

    dn = (((1,), (1,)), ((), ()))
    q = jax.lax.dot_general(x, wq_ref[0], dn,
                            preferred_element_type=jnp.float32)
    k = jax.lax.dot_general(x, wk_ref[0], dn,
                            preferred_element_type=jnp.float32)
    v = jax.lax.dot_general(x, wv_ref[0], dn,
                            preferred_element_type=jnp.float32)

    def rope(z):
        z1 = z[:, : _HD // 2]
        z2 = z[:, _HD // 2:]
        rz = jnp.concatenate([-z2, z1], axis=-1)
        return z * cos + rz * sin

    q = rope(q)
    k = rope(k)

    s = jax.lax.dot_general(q, k, dn,
                            preferred_element_type=jnp.float32) * _SCALE
    row = jax.lax.broadcasted_iota(jnp.int32, (_S, _S), 0)
    col = jax.lax.broadcasted_iota(jnp.int32, (_S, _S), 1)
    s = jnp.where(col <= row, s, _NEG)
    m = jnp.max(s, axis=-1, keepdims=True)
    p = jnp.exp(s - m)
    p = p / jnp.sum(p, axis=-1, keepdims=True)

    o = jnp.dot(p, v, preferred_element_type=jnp.float32)
    partial = jax.lax.dot_general(o, wo_ref[0], dn,
                                  preferred_element_type=jnp.float32)

    @pl.when(h == 0)
    def _():
        out_ref[...] = partial

    @pl.when(h > 0)
    def _():
        out_ref[...] += partial


def kernel(hidden_states, position_ids, Wq, Wk, Wv, Wo):
    x = hidden_states[0]
    pos = position_ids[0].astype(jnp.float32)
    inv_freq = 1.0 / (_THETA ** (jnp.arange(0, _HD, 2, dtype=jnp.float32) / _HD))
    freqs = pos[:, None] * inv_freq[None, :]
    emb = jnp.concatenate([freqs, freqs], axis=-1)
    cos = jnp.cos(emb)
    sin = jnp.sin(emb)

    wq_r = Wq.reshape(_NH, _HD, _HID)
    wk_r = Wk.reshape(_NH, _HD, _HID)
    wv_r = Wv.reshape(_NH, _HD, _HID)
    wo_r = Wo.reshape(_HID, _NH, _HD).transpose(1, 0, 2)

    const = lambda h: (0, 0)
    per_head2 = lambda h: (h, 0, 0)
    out = pl.pallas_call(
        _attn_head_kernel,
        grid=(_NH,),
        in_specs=[
            pl.BlockSpec((_S, _HID), const),
            pl.BlockSpec((_S, _HD), const),
            pl.BlockSpec((_S, _HD), const),
            pl.BlockSpec((1, _HD, _HID), per_head2),
            pl.BlockSpec((1, _HD, _HID), per_head2),
            pl.BlockSpec((1, _HD, _HID), per_head2),
            pl.BlockSpec((1, _HID, _HD), per_head2),
        ],
        out_specs=pl.BlockSpec((_S, _HID), const),
        out_shape=jax.ShapeDtypeStruct((_S, _HID), jnp.float32),
    )(x, cos, sin, wq_r, wk_r, wv_r, wo_r)
    return out[None]
